# Initial kernel scaffold; baseline (speedup 1.0000x reference)
import jax
import jax.numpy as jnp
from jax.experimental import pallas as pl


def kernel(stock_features, wiki_edge_index, wiki_efeat, industry_edge_index, industry_efeat, correlation_edge_index, correlation_efeat, lstm_params, wiki_params, industry_params, corr_params, attn_params, readout_params):
    raise NotImplementedError("write your pallas kernel here")


# speedup vs baseline: 6.5168x; 6.5168x over previous
import functools

import jax
import jax.numpy as jnp
from jax import lax
from jax.experimental import pallas as pl
from jax.experimental.pallas import tpu as pltpu
from jax.experimental.pallas import tpu_sc as plsc

N = 10000
E = 160000
T = 8
D_IN = 5
H = 64

NCORE = 2
NSUB = 16
CHUNK = 128

_f32 = jnp.float32


def _leaky(x):
    return jnp.maximum(x, 0.2 * x)


def _lstm_body(xf_ref, wih_ref, whh_ref, b_ref, h_ref, xn0_ref):
    xf = xf_ref[...]
    mean = xf[:, 0:D_IN]
    for t in range(1, T):
        mean = mean + xf[:, t * D_IN:(t + 1) * D_IN]
    mean = mean * (1.0 / T)
    wih = wih_ref[...]
    whh = whh_ref[...]
    b = b_ref[...]
    nb = xf.shape[0]
    h = jnp.zeros((nb, H), _f32)
    c = jnp.zeros((nb, H), _f32)
    for t in range(T):
        xt = xf[:, t * D_IN:(t + 1) * D_IN] / mean
        g = (jnp.dot(xt, wih, preferred_element_type=_f32)
             + jnp.dot(h, whh, preferred_element_type=_f32) + b)
        i = jax.nn.sigmoid(g[:, 0:H])
        f = jax.nn.sigmoid(g[:, H:2 * H])
        gg = jnp.tanh(g[:, 2 * H:3 * H])
        o = jax.nn.sigmoid(g[:, 3 * H:4 * H])
        c = f * c + i * gg
        h = o * jnp.tanh(c)
        h_ref[t] = h
        if t == T - 1:
            xn0_ref[...] = xt[:, 0:1]


def _run_lstm(xf, wih, whh, b):
    nb = 1000
    grid = (N // nb,)
    return pl.pallas_call(
        _lstm_body,
        grid=grid,
        in_specs=[
            pl.BlockSpec((nb, T * D_IN), lambda i: (i, 0)),
            pl.BlockSpec((D_IN, 4 * H), lambda i: (0, 0)),
            pl.BlockSpec((H, 4 * H), lambda i: (0, 0)),
            pl.BlockSpec((1, 4 * H), lambda i: (0, 0)),
        ],
        out_specs=[
            pl.BlockSpec((T, nb, H), lambda i: (0, i, 0)),
            pl.BlockSpec((nb, 1), lambda i: (i, 0)),
        ],
        out_shape=[
            jax.ShapeDtypeStruct((T, N, H), _f32),
            jax.ShapeDtypeStruct((N, 1), _f32),
        ],
    )(xf, wih, whh, b)


T2 = T // 2


def _proj_body(h_ref, w_ref, b_ref, *out_refs):
    h0 = h_ref[0]
    h1 = h_ref[1]
    w = w_ref[...]
    b = b_ref[...]
    for j in range(6):
        a = jnp.dot(h0, w[j], preferred_element_type=_f32) + b[j]
        bb = jnp.dot(h1, w[j], preferred_element_type=_f32) + b[j]
        out_refs[j][0] = jnp.concatenate([a, bb], axis=1)


def _run_proj(h, w6, b6):
    nb = 2000
    grid = (T2, N // nb)
    return pl.pallas_call(
        _proj_body,
        grid=grid,
        in_specs=[
            pl.BlockSpec((2, nb, H), lambda t, i: (t, i, 0)),
            pl.BlockSpec((6, H, H), lambda t, i: (0, 0, 0)),
            pl.BlockSpec((6, 1, H), lambda t, i: (0, 0, 0)),
        ],
        out_specs=[pl.BlockSpec((1, nb, 2 * H), lambda t, i: (t, i, 0))] * 6,
        out_shape=[jax.ShapeDtypeStruct((T2, N, 2 * H), _f32)] * 6,
    )(h, w6, b6)


def _edge_body(fw_ref, fi_ref, fc_ref, w_ref, b_ref, ow_ref, oi_ref, oc_ref):
    w = w_ref[...]
    b = b_ref[...]
    for g, (fr, orf) in enumerate(((fw_ref, ow_ref), (fi_ref, oi_ref),
                                   (fc_ref, oc_ref))):
        orf[...] = jnp.dot(fr[...], w[g], preferred_element_type=_f32) + b[g]


def _run_edge(fw, fi, fc, w3, b3):
    eb = 2000
    grid = (E // eb,)
    fspec = pl.BlockSpec((eb, fw.shape[1]), lambda i: (i, 0))
    ospec = pl.BlockSpec((eb, H), lambda i: (i, 0))
    return pl.pallas_call(
        _edge_body,
        grid=grid,
        in_specs=[
            fspec, fspec, fspec,
            pl.BlockSpec((3, fw.shape[1], H), lambda i: (0, 0, 0)),
            pl.BlockSpec((3, 1, H), lambda i: (0, 0, 0)),
        ],
        out_specs=[ospec] * 3,
        out_shape=[jax.ShapeDtypeStruct((E, H), _f32)] * 3,
    )(fw, fi, fc, w3, b3)


def _sc_body(eq, ek, e, src, dst, zz, out,
             acc, srcv, dstv, eqr, ekr, er, sem1, sem2):
    c = lax.axis_index("c")
    s = lax.axis_index("s")
    r_main = (N // NSUB) // 8 * 8
    r_last = N - (NSUB - 1) * r_main
    chunks_per_core = (E // CHUNK) // NCORE
    iters = (chunks_per_core + NSUB - 1) // NSUB

    def _rowcopy(src_ref, dst_ref):
        @pl.when(s < NSUB - 1)
        def _():
            pltpu.sync_copy(src_ref.at[pl.ds(s * r_main, r_main)],
                            dst_ref.at[pl.ds(s * r_main, r_main)])

        @pl.when(s == NSUB - 1)
        def _():
            base = (NSUB - 1) * r_main
            pltpu.sync_copy(src_ref.at[pl.ds(base, r_last)],
                            dst_ref.at[pl.ds(base, r_last)])

    for tp in range(T2):
        _rowcopy(zz, acc)
        plsc.subcore_barrier()

        def chunk_body(it, _, tp=tp):
            ch = s + it * NSUB

            @pl.when(ch < chunks_per_core)
            def _():
                base = (c * chunks_per_core + ch) * CHUNK
                pltpu.sync_copy(src.at[pl.ds(base, CHUNK)], srcv)
                pltpu.sync_copy(dst.at[pl.ds(base, CHUNK)], dstv)
                pltpu.sync_copy(e.at[pl.ds(base, CHUNK)], er)
                cp1 = pltpu.async_copy(eq.at[tp].at[dstv], eqr, sem1)
                cp2 = pltpu.async_copy(ek.at[tp].at[srcv], ekr, sem2)
                cp1.wait()
                cp2.wait()

                def row_body(r, _):
                    for cc in range(2 * H // 16):
                        sl = pl.ds(cc * 16, 16)
                        esl = pl.ds(cc % (H // 16) * 16, 16)
                        v = eqr[r, sl] + ekr[r, sl] + er[r, esl]
                        eqr[r, sl] = jnp.maximum(v, 0.2 * v)
                    return 0

                lax.fori_loop(0, CHUNK, row_body, 0)
                pltpu.sync_copy(eqr, acc.at[dstv], add=True)
            return 0

        lax.fori_loop(0, iters, chunk_body, 0)
        plsc.subcore_barrier()
        _rowcopy(acc, out.at[c].at[tp])
        plsc.subcore_barrier()


def _make_sc_kernel():
    mesh = plsc.VectorSubcoreMesh(core_axis_name="c", subcore_axis_name="s",
                                  num_cores=NCORE, num_subcores=NSUB)
    return pl.kernel(
        _sc_body,
        out_type=jax.ShapeDtypeStruct((NCORE, T2, N, 2 * H), _f32),
        mesh=mesh,
        scratch_types=[
            pltpu.VMEM_SHARED((N, 2 * H), _f32),
            pltpu.VMEM((CHUNK,), jnp.int32),
            pltpu.VMEM((CHUNK,), jnp.int32),
            pltpu.VMEM((CHUNK, 2 * H), _f32),
            pltpu.VMEM((CHUNK, 2 * H), _f32),
            pltpu.VMEM((CHUNK, H), _f32),
            pltpu.SemaphoreType.DMA,
            pltpu.SemaphoreType.DMA,
        ],
    )


def _readout_body(h_ref, fw_ref, fi_ref, fc_ref, xn0_ref,
                  wr_ref, br_ref, wa_ref, ba_ref, w1_ref, b1_ref,
                  w2_ref, b2_ref, out_ref):
    h = h_ref[...]
    sources = [[h[t] for t in range(T)]]
    for g, fref in enumerate((fw_ref, fi_ref, fc_ref)):
        f = fref[...]
        ft = f[0] + f[1]
        wr = wr_ref[g]
        br = br_ref[g]
        sources.append([
            jnp.dot(ft[t // 2][:, (t % 2) * H:(t % 2 + 1) * H], wr,
                    preferred_element_type=_f32) + br
            for t in range(T)])
    wa = wa_ref[...]
    ba = ba_ref[...]
    logit_cols = []
    for t in range(T):
        v = ba
        for p in range(4):
            v = v + jnp.dot(sources[p][t], wa[p * H:(p + 1) * H, :],
                            preferred_element_type=_f32)
        logit_cols.append(v)
    logits = jnp.concatenate(logit_cols, axis=1)
    m = jnp.max(logits, axis=1, keepdims=True)
    ex = jnp.exp(logits - m)
    w = ex / jnp.sum(ex, axis=1, keepdims=True)
    w1 = w1_ref[...]
    r = b1_ref[...]
    for p in range(4):
        pooled = sources[p][0] * w[:, 0:1]
        for t in range(1, T):
            pooled = pooled + sources[p][t] * w[:, t:t + 1]
        r = r + jnp.dot(pooled, w1[p * H:(p + 1) * H, :],
                        preferred_element_type=_f32)
    r = _leaky(r)
    r2 = _leaky(jnp.dot(r, w2_ref[...], preferred_element_type=_f32)
                + b2_ref[...])
    out_ref[...] = r2 / xn0_ref[...] - 1.0


def _run_readout(h, ftw, fti, ftc, xn0, wr3, br3, wa, ba, w1, b1, w2, b2):
    nb = 200
    grid = (N // nb,)
    fspec = pl.BlockSpec((NCORE, T2, nb, 2 * H), lambda i: (0, 0, i, 0))
    return pl.pallas_call(
        _readout_body,
        grid=grid,
        in_specs=[
            pl.BlockSpec((T, nb, H), lambda i: (0, i, 0)),
            fspec, fspec, fspec,
            pl.BlockSpec((nb, 1), lambda i: (i, 0)),
            pl.BlockSpec((3, H, H), lambda i: (0, 0, 0)),
            pl.BlockSpec((3, 1, H), lambda i: (0, 0, 0)),
            pl.BlockSpec((4 * H, 1), lambda i: (0, 0)),
            pl.BlockSpec((1, 1), lambda i: (0, 0)),
            pl.BlockSpec((4 * H, H), lambda i: (0, 0)),
            pl.BlockSpec((1, H), lambda i: (0, 0)),
            pl.BlockSpec((H, 1), lambda i: (0, 0)),
            pl.BlockSpec((1, 1), lambda i: (0, 0)),
        ],
        out_specs=pl.BlockSpec((nb, 1), lambda i: (i, 0)),
        out_shape=jax.ShapeDtypeStruct((N, 1), _f32),
    )(h, ftw, fti, ftc, xn0, wr3, br3, wa, ba, w1, b1, w2, b2)


def kernel(stock_features, wiki_edge_index, wiki_efeat, industry_edge_index,
           industry_efeat, correlation_edge_index, correlation_efeat,
           lstm_params, wiki_params, industry_params, corr_params,
           attn_params, readout_params):
    xf = stock_features.reshape(N, T * D_IN)
    wih = lstm_params["W_ih"].T
    whh = lstm_params["W_hh"].T
    b = (lstm_params["b_ih"] + lstm_params["b_hh"])[None, :]
    h, xn0 = _run_lstm(xf, wih, whh, b)

    gp = (wiki_params, industry_params, corr_params)
    wqk = jnp.stack([p[k].T for p in gp for k in ("Wq", "Wk")])
    bqk = jnp.stack([p[k][None, :] for p in gp for k in ("bq", "bk")])
    eqw, ekw, eqi, eki, eqc, ekc = _run_proj(h, wqk, bqk)

    we3 = jnp.stack([p["We"].T for p in gp])
    be3 = jnp.stack([p["be"][None, :] for p in gp])
    ew, ei, ec = _run_edge(wiki_efeat, industry_efeat, correlation_efeat,
                           we3, be3)

    zz = jnp.zeros((N, 2 * H), _f32)
    sc = _make_sc_kernel()
    ftw = sc(eqw, ekw, ew, wiki_edge_index[0], wiki_edge_index[1], zz)
    fti = sc(eqi, eki, ei, industry_edge_index[0], industry_edge_index[1], zz)
    ftc = sc(eqc, ekc, ec, correlation_edge_index[0],
             correlation_edge_index[1], zz)

    wr3 = jnp.stack([p["Wr"].T for p in gp])
    br3 = jnp.stack([p["br"][None, :] for p in gp])
    return _run_readout(
        h, ftw, fti, ftc, xn0, wr3, br3,
        attn_params["Wa"].T, attn_params["ba"][None, :],
        readout_params["W1"].T, readout_params["b1"][None, :],
        readout_params["W2"].T, readout_params["b2"][None, :])


# speedup vs baseline: 9.1211x; 1.3996x over previous
import functools

import jax
import jax.numpy as jnp
from jax import lax
from jax.experimental import pallas as pl
from jax.experimental.pallas import tpu as pltpu
from jax.experimental.pallas import tpu_sc as plsc

N = 10000
E = 160000
T = 8
D_IN = 5
H = 64

NCORE = 2
NSUB = 16
CHUNK = 64

_f32 = jnp.float32


def _leaky(x):
    return jnp.maximum(x, 0.2 * x)


def _lstm_body(xf_ref, wih_ref, whh_ref, b_ref, h_ref, xn0_ref):
    xf = xf_ref[...]
    mean = xf[:, 0:D_IN]
    for t in range(1, T):
        mean = mean + xf[:, t * D_IN:(t + 1) * D_IN]
    mean = mean * (1.0 / T)
    wih = wih_ref[...]
    whh = whh_ref[...]
    b = b_ref[...]
    nb = xf.shape[0]
    h = jnp.zeros((nb, H), _f32)
    c = jnp.zeros((nb, H), _f32)
    for t in range(T):
        xt = xf[:, t * D_IN:(t + 1) * D_IN] / mean
        g = (jnp.dot(xt, wih, preferred_element_type=_f32)
             + jnp.dot(h, whh, preferred_element_type=_f32) + b)
        i = jax.nn.sigmoid(g[:, 0:H])
        f = jax.nn.sigmoid(g[:, H:2 * H])
        gg = jnp.tanh(g[:, 2 * H:3 * H])
        o = jax.nn.sigmoid(g[:, 3 * H:4 * H])
        c = f * c + i * gg
        h = o * jnp.tanh(c)
        h_ref[t] = h
        if t == T - 1:
            xn0_ref[...] = xt[:, 0:1]


def _run_lstm(xf, wih, whh, b):
    nb = 1000
    grid = (N // nb,)
    return pl.pallas_call(
        _lstm_body,
        grid=grid,
        in_specs=[
            pl.BlockSpec((nb, T * D_IN), lambda i: (i, 0)),
            pl.BlockSpec((D_IN, 4 * H), lambda i: (0, 0)),
            pl.BlockSpec((H, 4 * H), lambda i: (0, 0)),
            pl.BlockSpec((1, 4 * H), lambda i: (0, 0)),
        ],
        out_specs=[
            pl.BlockSpec((T, nb, H), lambda i: (0, i, 0)),
            pl.BlockSpec((nb, 1), lambda i: (i, 0)),
        ],
        out_shape=[
            jax.ShapeDtypeStruct((T, N, H), _f32),
            jax.ShapeDtypeStruct((N, 1), _f32),
        ],
    )(xf, wih, whh, b)


T2 = T // 2


def _proj_body(h_ref, w_ref, b_ref, *out_refs):
    h0 = h_ref[0]
    h1 = h_ref[1]
    w = w_ref[...]
    b = b_ref[...]
    for j in range(6):
        a = jnp.dot(h0, w[j], preferred_element_type=_f32) + b[j]
        bb = jnp.dot(h1, w[j], preferred_element_type=_f32) + b[j]
        out_refs[j][0] = jnp.concatenate([a, bb], axis=1)


def _run_proj(h, w6, b6):
    nb = 2000
    grid = (T2, N // nb)
    return pl.pallas_call(
        _proj_body,
        grid=grid,
        in_specs=[
            pl.BlockSpec((2, nb, H), lambda t, i: (t, i, 0)),
            pl.BlockSpec((6, H, H), lambda t, i: (0, 0, 0)),
            pl.BlockSpec((6, 1, H), lambda t, i: (0, 0, 0)),
        ],
        out_specs=[pl.BlockSpec((1, nb, 2 * H), lambda t, i: (t, i, 0))] * 6,
        out_shape=[jax.ShapeDtypeStruct((T2, N, 2 * H), _f32)] * 6,
    )(h, w6, b6)


def _edge_body(fw_ref, fi_ref, fc_ref, w_ref, b_ref, ow_ref, oi_ref, oc_ref):
    w = w_ref[...]
    b = b_ref[...]
    for g, (fr, orf) in enumerate(((fw_ref, ow_ref), (fi_ref, oi_ref),
                                   (fc_ref, oc_ref))):
        orf[...] = jnp.dot(fr[...], w[g], preferred_element_type=_f32) + b[g]


def _run_edge(fw, fi, fc, w3, b3):
    eb = 2000
    grid = (E // eb,)
    fspec = pl.BlockSpec((eb, fw.shape[1]), lambda i: (i, 0))
    ospec = pl.BlockSpec((eb, H), lambda i: (i, 0))
    return pl.pallas_call(
        _edge_body,
        grid=grid,
        in_specs=[
            fspec, fspec, fspec,
            pl.BlockSpec((3, fw.shape[1], H), lambda i: (0, 0, 0)),
            pl.BlockSpec((3, 1, H), lambda i: (0, 0, 0)),
        ],
        out_specs=[ospec] * 3,
        out_shape=[jax.ShapeDtypeStruct((E, H), _f32)] * 3,
    )(fw, fi, fc, w3, b3)


def _sc_body(eq, ek, e, src, dst, zz, out,
             acc, srcv, dstv, eqr, ekr, er,
             semq0, semk0, seme0, semq1, semk1, seme1):
    sems = ((semq0, semk0, seme0), (semq1, semk1, seme1))
    c = lax.axis_index("c")
    s = lax.axis_index("s")
    r_main = (N // NSUB) // 8 * 8
    r_last = N - (NSUB - 1) * r_main
    chunks_per_core = (E // CHUNK) // NCORE
    iters = (chunks_per_core + NSUB - 1) // NSUB
    iters = (iters + 1) // 2 * 2

    def _rowcopy(src_ref, dst_ref):
        @pl.when(s < NSUB - 1)
        def _():
            pltpu.sync_copy(src_ref.at[pl.ds(s * r_main, r_main)],
                            dst_ref.at[pl.ds(s * r_main, r_main)])

        @pl.when(s == NSUB - 1)
        def _():
            base = (NSUB - 1) * r_main
            pltpu.sync_copy(src_ref.at[pl.ds(base, r_last)],
                            dst_ref.at[pl.ds(base, r_last)])

    def _prefetch(b, it, tp):
        ch = s + it * NSUB

        @pl.when(ch < chunks_per_core)
        def _():
            base = (c * chunks_per_core + ch) * CHUNK
            pltpu.sync_copy(src.at[pl.ds(base, CHUNK)], srcv.at[b])
            pltpu.sync_copy(dst.at[pl.ds(base, CHUNK)], dstv.at[b])
            pltpu.async_copy(e.at[pl.ds(base, CHUNK)], er.at[b], sems[b][2])
            pltpu.async_copy(eq.at[tp].at[dstv.at[b]], eqr.at[b], sems[b][0])
            pltpu.async_copy(ek.at[tp].at[srcv.at[b]], ekr.at[b], sems[b][1])

    def _consume(b, it, tp):
        ch = s + it * NSUB

        @pl.when(ch < chunks_per_core)
        def _():
            pltpu.make_async_copy(eq.at[tp].at[dstv.at[b]], eqr.at[b],
                                  sems[b][0]).wait()
            pltpu.make_async_copy(ek.at[tp].at[srcv.at[b]], ekr.at[b],
                                  sems[b][1]).wait()
            pltpu.make_async_copy(e.at[pl.ds(0, CHUNK)], er.at[b],
                                  sems[b][2]).wait()

            def row_body(r, _):
                for cc in range(2 * H // 16):
                    sl = pl.ds(cc * 16, 16)
                    esl = pl.ds(cc % (H // 16) * 16, 16)
                    v = eqr[b, r, sl] + ekr[b, r, sl] + er[b, r, esl]
                    eqr[b, r, sl] = jnp.maximum(v, 0.2 * v)
                return 0

            lax.fori_loop(0, CHUNK, row_body, 0)
            pltpu.sync_copy(eqr.at[b], acc.at[dstv.at[b]], add=True)

    for tp in range(T2):
        _rowcopy(zz, acc)
        plsc.subcore_barrier()

        _prefetch(0, 0, tp)

        def pair_body(it2, _, tp=tp):
            it = it2 * 2
            _prefetch(1, it + 1, tp)
            _consume(0, it, tp)
            _prefetch(0, it + 2, tp)
            _consume(1, it + 1, tp)
            return 0

        lax.fori_loop(0, iters // 2, pair_body, 0)
        plsc.subcore_barrier()
        _rowcopy(acc, out.at[c].at[tp])
        plsc.subcore_barrier()


def _make_sc_kernel():
    mesh = plsc.VectorSubcoreMesh(core_axis_name="c", subcore_axis_name="s",
                                  num_cores=NCORE, num_subcores=NSUB)
    return pl.kernel(
        _sc_body,
        out_type=jax.ShapeDtypeStruct((NCORE, T2, N, 2 * H), _f32),
        mesh=mesh,
        scratch_types=[
            pltpu.VMEM_SHARED((N, 2 * H), _f32),
            pltpu.VMEM((2, CHUNK), jnp.int32),
            pltpu.VMEM((2, CHUNK), jnp.int32),
            pltpu.VMEM((2, CHUNK, 2 * H), _f32),
            pltpu.VMEM((2, CHUNK, 2 * H), _f32),
            pltpu.VMEM((2, CHUNK, H), _f32),
            pltpu.SemaphoreType.DMA,
            pltpu.SemaphoreType.DMA,
            pltpu.SemaphoreType.DMA,
            pltpu.SemaphoreType.DMA,
            pltpu.SemaphoreType.DMA,
            pltpu.SemaphoreType.DMA,
        ],
    )


def _readout_body(h_ref, fw_ref, fi_ref, fc_ref, xn0_ref,
                  wr_ref, br_ref, wa_ref, ba_ref, w1_ref, b1_ref,
                  w2_ref, b2_ref, out_ref):
    h = h_ref[...]
    sources = [[h[t] for t in range(T)]]
    for g, fref in enumerate((fw_ref, fi_ref, fc_ref)):
        f = fref[...]
        ft = f[0] + f[1]
        wr = wr_ref[g]
        br = br_ref[g]
        sources.append([
            jnp.dot(ft[t // 2][:, (t % 2) * H:(t % 2 + 1) * H], wr,
                    preferred_element_type=_f32) + br
            for t in range(T)])
    wa = wa_ref[...]
    ba = ba_ref[...]
    logit_cols = []
    for t in range(T):
        v = ba
        for p in range(4):
            v = v + jnp.dot(sources[p][t], wa[p * H:(p + 1) * H, :],
                            preferred_element_type=_f32)
        logit_cols.append(v)
    logits = jnp.concatenate(logit_cols, axis=1)
    m = jnp.max(logits, axis=1, keepdims=True)
    ex = jnp.exp(logits - m)
    w = ex / jnp.sum(ex, axis=1, keepdims=True)
    w1 = w1_ref[...]
    r = b1_ref[...]
    for p in range(4):
        pooled = sources[p][0] * w[:, 0:1]
        for t in range(1, T):
            pooled = pooled + sources[p][t] * w[:, t:t + 1]
        r = r + jnp.dot(pooled, w1[p * H:(p + 1) * H, :],
                        preferred_element_type=_f32)
    r = _leaky(r)
    r2 = _leaky(jnp.dot(r, w2_ref[...], preferred_element_type=_f32)
                + b2_ref[...])
    out_ref[...] = r2 / xn0_ref[...] - 1.0


def _run_readout(h, ftw, fti, ftc, xn0, wr3, br3, wa, ba, w1, b1, w2, b2):
    nb = 200
    grid = (N // nb,)
    fspec = pl.BlockSpec((NCORE, T2, nb, 2 * H), lambda i: (0, 0, i, 0))
    return pl.pallas_call(
        _readout_body,
        grid=grid,
        in_specs=[
            pl.BlockSpec((T, nb, H), lambda i: (0, i, 0)),
            fspec, fspec, fspec,
            pl.BlockSpec((nb, 1), lambda i: (i, 0)),
            pl.BlockSpec((3, H, H), lambda i: (0, 0, 0)),
            pl.BlockSpec((3, 1, H), lambda i: (0, 0, 0)),
            pl.BlockSpec((4 * H, 1), lambda i: (0, 0)),
            pl.BlockSpec((1, 1), lambda i: (0, 0)),
            pl.BlockSpec((4 * H, H), lambda i: (0, 0)),
            pl.BlockSpec((1, H), lambda i: (0, 0)),
            pl.BlockSpec((H, 1), lambda i: (0, 0)),
            pl.BlockSpec((1, 1), lambda i: (0, 0)),
        ],
        out_specs=pl.BlockSpec((nb, 1), lambda i: (i, 0)),
        out_shape=jax.ShapeDtypeStruct((N, 1), _f32),
    )(h, ftw, fti, ftc, xn0, wr3, br3, wa, ba, w1, b1, w2, b2)


def kernel(stock_features, wiki_edge_index, wiki_efeat, industry_edge_index,
           industry_efeat, correlation_edge_index, correlation_efeat,
           lstm_params, wiki_params, industry_params, corr_params,
           attn_params, readout_params):
    xf = stock_features.reshape(N, T * D_IN)
    wih = lstm_params["W_ih"].T
    whh = lstm_params["W_hh"].T
    b = (lstm_params["b_ih"] + lstm_params["b_hh"])[None, :]
    h, xn0 = _run_lstm(xf, wih, whh, b)

    gp = (wiki_params, industry_params, corr_params)
    wqk = jnp.stack([p[k].T for p in gp for k in ("Wq", "Wk")])
    bqk = jnp.stack([p[k][None, :] for p in gp for k in ("bq", "bk")])
    eqw, ekw, eqi, eki, eqc, ekc = _run_proj(h, wqk, bqk)

    we3 = jnp.stack([p["We"].T for p in gp])
    be3 = jnp.stack([p["be"][None, :] for p in gp])
    ew, ei, ec = _run_edge(wiki_efeat, industry_efeat, correlation_efeat,
                           we3, be3)

    zz = jnp.zeros((N, 2 * H), _f32)
    sc = _make_sc_kernel()
    ftw = sc(eqw, ekw, ew, wiki_edge_index[0], wiki_edge_index[1], zz)
    fti = sc(eqi, eki, ei, industry_edge_index[0], industry_edge_index[1], zz)
    ftc = sc(eqc, ekc, ec, correlation_edge_index[0],
             correlation_edge_index[1], zz)

    wr3 = jnp.stack([p["Wr"].T for p in gp])
    br3 = jnp.stack([p["br"][None, :] for p in gp])
    return _run_readout(
        h, ftw, fti, ftc, xn0, wr3, br3,
        attn_params["Wa"].T, attn_params["ba"][None, :],
        readout_params["W1"].T, readout_params["b1"][None, :],
        readout_params["W2"].T, readout_params["b2"][None, :])


# speedup vs baseline: 9.2358x; 1.0126x over previous
import functools

import jax
import jax.numpy as jnp
from jax import lax
from jax.experimental import pallas as pl
from jax.experimental.pallas import tpu as pltpu
from jax.experimental.pallas import tpu_sc as plsc

N = 10000
E = 160000
T = 8
D_IN = 5
H = 64

NCORE = 2
NSUB = 16
CHUNK = 64

_f32 = jnp.float32


def _leaky(x):
    return jnp.maximum(x, 0.2 * x)


def _lstm_body(xf_ref, wih_ref, whh_ref, b_ref, h_ref, xn0_ref):
    xf = xf_ref[...]
    mean = xf[:, 0:D_IN]
    for t in range(1, T):
        mean = mean + xf[:, t * D_IN:(t + 1) * D_IN]
    mean = mean * (1.0 / T)
    wih = wih_ref[...]
    whh = whh_ref[...]
    b = b_ref[...]
    nb = xf.shape[0]
    h = jnp.zeros((nb, H), _f32)
    c = jnp.zeros((nb, H), _f32)
    for t in range(T):
        xt = xf[:, t * D_IN:(t + 1) * D_IN] / mean
        g = (jnp.dot(xt, wih, preferred_element_type=_f32)
             + jnp.dot(h, whh, preferred_element_type=_f32) + b)
        i = jax.nn.sigmoid(g[:, 0:H])
        f = jax.nn.sigmoid(g[:, H:2 * H])
        gg = jnp.tanh(g[:, 2 * H:3 * H])
        o = jax.nn.sigmoid(g[:, 3 * H:4 * H])
        c = f * c + i * gg
        h = o * jnp.tanh(c)
        h_ref[t] = h
        if t == T - 1:
            xn0_ref[...] = xt[:, 0:1]


def _run_lstm(xf, wih, whh, b):
    nb = 1000
    grid = (N // nb,)
    return pl.pallas_call(
        _lstm_body,
        grid=grid,
        in_specs=[
            pl.BlockSpec((nb, T * D_IN), lambda i: (i, 0)),
            pl.BlockSpec((D_IN, 4 * H), lambda i: (0, 0)),
            pl.BlockSpec((H, 4 * H), lambda i: (0, 0)),
            pl.BlockSpec((1, 4 * H), lambda i: (0, 0)),
        ],
        out_specs=[
            pl.BlockSpec((T, nb, H), lambda i: (0, i, 0)),
            pl.BlockSpec((nb, 1), lambda i: (i, 0)),
        ],
        out_shape=[
            jax.ShapeDtypeStruct((T, N, H), _f32),
            jax.ShapeDtypeStruct((N, 1), _f32),
        ],
    )(xf, wih, whh, b)


T2 = T // 2


def _proj_body(h_ref, w_ref, b_ref, *out_refs):
    h0 = h_ref[0]
    h1 = h_ref[1]
    w = w_ref[...]
    b = b_ref[...]
    for j in range(6):
        a = jnp.dot(h0, w[j], preferred_element_type=_f32) + b[j]
        bb = jnp.dot(h1, w[j], preferred_element_type=_f32) + b[j]
        out_refs[j][0] = jnp.concatenate([a, bb], axis=1)


def _run_proj(h, w6, b6):
    nb = 2000
    grid = (T2, N // nb)
    return pl.pallas_call(
        _proj_body,
        grid=grid,
        in_specs=[
            pl.BlockSpec((2, nb, H), lambda t, i: (t, i, 0)),
            pl.BlockSpec((6, H, H), lambda t, i: (0, 0, 0)),
            pl.BlockSpec((6, 1, H), lambda t, i: (0, 0, 0)),
        ],
        out_specs=[pl.BlockSpec((1, nb, 2 * H), lambda t, i: (t, i, 0))] * 6,
        out_shape=[jax.ShapeDtypeStruct((T2, N, 2 * H), _f32)] * 6,
    )(h, w6, b6)


def _edge_body(fw_ref, fi_ref, fc_ref, w_ref, b_ref, ow_ref, oi_ref, oc_ref):
    w = w_ref[...]
    b = b_ref[...]
    for g, (fr, orf) in enumerate(((fw_ref, ow_ref), (fi_ref, oi_ref),
                                   (fc_ref, oc_ref))):
        orf[...] = jnp.dot(fr[...], w[g], preferred_element_type=_f32) + b[g]


def _run_edge(fw, fi, fc, w3, b3):
    eb = 2000
    grid = (E // eb,)
    fspec = pl.BlockSpec((eb, fw.shape[1]), lambda i: (i, 0))
    ospec = pl.BlockSpec((eb, H), lambda i: (i, 0))
    return pl.pallas_call(
        _edge_body,
        grid=grid,
        in_specs=[
            fspec, fspec, fspec,
            pl.BlockSpec((3, fw.shape[1], H), lambda i: (0, 0, 0)),
            pl.BlockSpec((3, 1, H), lambda i: (0, 0, 0)),
        ],
        out_specs=[ospec] * 3,
        out_shape=[jax.ShapeDtypeStruct((E, H), _f32)] * 3,
    )(fw, fi, fc, w3, b3)


def _sc_body(eq, ek, e, src, dst, zz, out,
             acc, srcv, dstv, eqr, ekr, er,
             semq0, semk0, seme0, semq1, semk1, seme1):
    sems = ((semq0, semk0, seme0), (semq1, semk1, seme1))
    c = lax.axis_index("c")
    s = lax.axis_index("s")
    r_main = (N // NSUB) // 8 * 8
    r_last = N - (NSUB - 1) * r_main
    chunks_per_core = (E // CHUNK) // NCORE
    iters = (chunks_per_core + NSUB - 1) // NSUB
    iters = (iters + 1) // 2 * 2

    def _rowcopy(src_ref, dst_ref):
        @pl.when(s < NSUB - 1)
        def _():
            pltpu.sync_copy(src_ref.at[pl.ds(s * r_main, r_main)],
                            dst_ref.at[pl.ds(s * r_main, r_main)])

        @pl.when(s == NSUB - 1)
        def _():
            base = (NSUB - 1) * r_main
            pltpu.sync_copy(src_ref.at[pl.ds(base, r_last)],
                            dst_ref.at[pl.ds(base, r_last)])

    def _prefetch(b, it, tp):
        ch = s + it * NSUB

        @pl.when(ch < chunks_per_core)
        def _():
            base = (c * chunks_per_core + ch) * CHUNK
            pltpu.sync_copy(src.at[pl.ds(base, CHUNK)], srcv.at[b])
            pltpu.sync_copy(dst.at[pl.ds(base, CHUNK)], dstv.at[b])
            pltpu.async_copy(e.at[pl.ds(base, CHUNK)], er.at[b], sems[b][2])
            pltpu.async_copy(eq.at[tp].at[dstv.at[b]], eqr.at[b], sems[b][0])
            pltpu.async_copy(ek.at[tp].at[srcv.at[b]], ekr.at[b], sems[b][1])

    def _consume(b, it, tp):
        ch = s + it * NSUB

        @pl.when(ch < chunks_per_core)
        def _():
            pltpu.make_async_copy(eq.at[tp].at[dstv.at[b]], eqr.at[b],
                                  sems[b][0]).wait()
            pltpu.make_async_copy(ek.at[tp].at[srcv.at[b]], ekr.at[b],
                                  sems[b][1]).wait()
            pltpu.make_async_copy(e.at[pl.ds(0, CHUNK)], er.at[b],
                                  sems[b][2]).wait()

            @plsc.parallel_loop(0, CHUNK, unroll=2)
            def row_body(r):
                for cc in range(H // 16):
                    ev = er[b, r, pl.ds(cc * 16, 16)]
                    for u in range(2):
                        sl = pl.ds(u * H + cc * 16, 16)
                        v = eqr[b, r, sl] + ekr[b, r, sl] + ev
                        eqr[b, r, sl] = jnp.maximum(v, 0.2 * v)
            pltpu.sync_copy(eqr.at[b], acc.at[dstv.at[b]], add=True)

    for tp in range(T2):
        _rowcopy(zz, acc)
        plsc.subcore_barrier()

        _prefetch(0, 0, tp)

        def pair_body(it2, _, tp=tp):
            it = it2 * 2
            _prefetch(1, it + 1, tp)
            _consume(0, it, tp)
            _prefetch(0, it + 2, tp)
            _consume(1, it + 1, tp)
            return 0

        lax.fori_loop(0, iters // 2, pair_body, 0)
        plsc.subcore_barrier()
        _rowcopy(acc, out.at[c].at[tp])
        plsc.subcore_barrier()


def _make_sc_kernel():
    mesh = plsc.VectorSubcoreMesh(core_axis_name="c", subcore_axis_name="s",
                                  num_cores=NCORE, num_subcores=NSUB)
    return pl.kernel(
        _sc_body,
        out_type=jax.ShapeDtypeStruct((NCORE, T2, N, 2 * H), _f32),
        mesh=mesh,
        scratch_types=[
            pltpu.VMEM_SHARED((N, 2 * H), _f32),
            pltpu.VMEM((2, CHUNK), jnp.int32),
            pltpu.VMEM((2, CHUNK), jnp.int32),
            pltpu.VMEM((2, CHUNK, 2 * H), _f32),
            pltpu.VMEM((2, CHUNK, 2 * H), _f32),
            pltpu.VMEM((2, CHUNK, H), _f32),
            pltpu.SemaphoreType.DMA,
            pltpu.SemaphoreType.DMA,
            pltpu.SemaphoreType.DMA,
            pltpu.SemaphoreType.DMA,
            pltpu.SemaphoreType.DMA,
            pltpu.SemaphoreType.DMA,
        ],
    )


def _readout_body(h_ref, fw_ref, fi_ref, fc_ref, xn0_ref,
                  wr_ref, br_ref, wa_ref, ba_ref, w1_ref, b1_ref,
                  w2_ref, b2_ref, out_ref):
    h = h_ref[...]
    sources = [[h[t] for t in range(T)]]
    for g, fref in enumerate((fw_ref, fi_ref, fc_ref)):
        f = fref[...]
        ft = f[0] + f[1]
        wr = wr_ref[g]
        br = br_ref[g]
        sources.append([
            jnp.dot(ft[t // 2][:, (t % 2) * H:(t % 2 + 1) * H], wr,
                    preferred_element_type=_f32) + br
            for t in range(T)])
    wa = wa_ref[...]
    ba = ba_ref[...]
    logit_cols = []
    for t in range(T):
        v = ba
        for p in range(4):
            v = v + jnp.dot(sources[p][t], wa[p * H:(p + 1) * H, :],
                            preferred_element_type=_f32)
        logit_cols.append(v)
    logits = jnp.concatenate(logit_cols, axis=1)
    m = jnp.max(logits, axis=1, keepdims=True)
    ex = jnp.exp(logits - m)
    w = ex / jnp.sum(ex, axis=1, keepdims=True)
    w1 = w1_ref[...]
    r = b1_ref[...]
    for p in range(4):
        pooled = sources[p][0] * w[:, 0:1]
        for t in range(1, T):
            pooled = pooled + sources[p][t] * w[:, t:t + 1]
        r = r + jnp.dot(pooled, w1[p * H:(p + 1) * H, :],
                        preferred_element_type=_f32)
    r = _leaky(r)
    r2 = _leaky(jnp.dot(r, w2_ref[...], preferred_element_type=_f32)
                + b2_ref[...])
    out_ref[...] = r2 / xn0_ref[...] - 1.0


def _run_readout(h, ftw, fti, ftc, xn0, wr3, br3, wa, ba, w1, b1, w2, b2):
    nb = 200
    grid = (N // nb,)
    fspec = pl.BlockSpec((NCORE, T2, nb, 2 * H), lambda i: (0, 0, i, 0))
    return pl.pallas_call(
        _readout_body,
        grid=grid,
        in_specs=[
            pl.BlockSpec((T, nb, H), lambda i: (0, i, 0)),
            fspec, fspec, fspec,
            pl.BlockSpec((nb, 1), lambda i: (i, 0)),
            pl.BlockSpec((3, H, H), lambda i: (0, 0, 0)),
            pl.BlockSpec((3, 1, H), lambda i: (0, 0, 0)),
            pl.BlockSpec((4 * H, 1), lambda i: (0, 0)),
            pl.BlockSpec((1, 1), lambda i: (0, 0)),
            pl.BlockSpec((4 * H, H), lambda i: (0, 0)),
            pl.BlockSpec((1, H), lambda i: (0, 0)),
            pl.BlockSpec((H, 1), lambda i: (0, 0)),
            pl.BlockSpec((1, 1), lambda i: (0, 0)),
        ],
        out_specs=pl.BlockSpec((nb, 1), lambda i: (i, 0)),
        out_shape=jax.ShapeDtypeStruct((N, 1), _f32),
    )(h, ftw, fti, ftc, xn0, wr3, br3, wa, ba, w1, b1, w2, b2)


def kernel(stock_features, wiki_edge_index, wiki_efeat, industry_edge_index,
           industry_efeat, correlation_edge_index, correlation_efeat,
           lstm_params, wiki_params, industry_params, corr_params,
           attn_params, readout_params):
    xf = stock_features.reshape(N, T * D_IN)
    wih = lstm_params["W_ih"].T
    whh = lstm_params["W_hh"].T
    b = (lstm_params["b_ih"] + lstm_params["b_hh"])[None, :]
    h, xn0 = _run_lstm(xf, wih, whh, b)

    gp = (wiki_params, industry_params, corr_params)
    wqk = jnp.stack([p[k].T for p in gp for k in ("Wq", "Wk")])
    bqk = jnp.stack([p[k][None, :] for p in gp for k in ("bq", "bk")])
    eqw, ekw, eqi, eki, eqc, ekc = _run_proj(h, wqk, bqk)

    we3 = jnp.stack([p["We"].T for p in gp])
    be3 = jnp.stack([p["be"][None, :] for p in gp])
    ew, ei, ec = _run_edge(wiki_efeat, industry_efeat, correlation_efeat,
                           we3, be3)

    zz = jnp.zeros((N, 2 * H), _f32)
    sc = _make_sc_kernel()
    ftw = sc(eqw, ekw, ew, wiki_edge_index[0], wiki_edge_index[1], zz)
    fti = sc(eqi, eki, ei, industry_edge_index[0], industry_edge_index[1], zz)
    ftc = sc(eqc, ekc, ec, correlation_edge_index[0],
             correlation_edge_index[1], zz)

    wr3 = jnp.stack([p["Wr"].T for p in gp])
    br3 = jnp.stack([p["br"][None, :] for p in gp])
    return _run_readout(
        h, ftw, fti, ftc, xn0, wr3, br3,
        attn_params["Wa"].T, attn_params["ba"][None, :],
        readout_params["W1"].T, readout_params["b1"][None, :],
        readout_params["W2"].T, readout_params["b2"][None, :])


# speedup vs baseline: 10.4420x; 1.1306x over previous
import functools

import jax
import jax.numpy as jnp
from jax import lax
from jax.experimental import pallas as pl
from jax.experimental.pallas import tpu as pltpu
from jax.experimental.pallas import tpu_sc as plsc

N = 10000
E = 160000
T = 8
D_IN = 5
H = 64

NCORE = 2
NSUB = 16
CHUNK = 32

_f32 = jnp.float32


def _leaky(x):
    return jnp.maximum(x, 0.2 * x)


def _lstm_body(xf_ref, wih_ref, whh_ref, b_ref, h_ref, xn0_ref):
    xf = xf_ref[...]
    mean = xf[:, 0:D_IN]
    for t in range(1, T):
        mean = mean + xf[:, t * D_IN:(t + 1) * D_IN]
    mean = mean * (1.0 / T)
    wih = wih_ref[...]
    whh = whh_ref[...]
    b = b_ref[...]
    nb = xf.shape[0]
    h = jnp.zeros((nb, H), _f32)
    c = jnp.zeros((nb, H), _f32)
    for t in range(T):
        xt = xf[:, t * D_IN:(t + 1) * D_IN] / mean
        g = (jnp.dot(xt, wih, preferred_element_type=_f32)
             + jnp.dot(h, whh, preferred_element_type=_f32) + b)
        i = jax.nn.sigmoid(g[:, 0:H])
        f = jax.nn.sigmoid(g[:, H:2 * H])
        gg = jnp.tanh(g[:, 2 * H:3 * H])
        o = jax.nn.sigmoid(g[:, 3 * H:4 * H])
        c = f * c + i * gg
        h = o * jnp.tanh(c)
        h_ref[t] = h
        if t == T - 1:
            xn0_ref[...] = xt[:, 0:1]


def _run_lstm(xf, wih, whh, b):
    nb = 1000
    grid = (N // nb,)
    return pl.pallas_call(
        _lstm_body,
        grid=grid,
        in_specs=[
            pl.BlockSpec((nb, T * D_IN), lambda i: (i, 0)),
            pl.BlockSpec((D_IN, 4 * H), lambda i: (0, 0)),
            pl.BlockSpec((H, 4 * H), lambda i: (0, 0)),
            pl.BlockSpec((1, 4 * H), lambda i: (0, 0)),
        ],
        out_specs=[
            pl.BlockSpec((T, nb, H), lambda i: (0, i, 0)),
            pl.BlockSpec((nb, 1), lambda i: (i, 0)),
        ],
        out_shape=[
            jax.ShapeDtypeStruct((T, N, H), _f32),
            jax.ShapeDtypeStruct((N, 1), _f32),
        ],
    )(xf, wih, whh, b)


T2 = T // 2


def _proj_body(h_ref, w_ref, b_ref, *out_refs):
    h0 = h_ref[0]
    h1 = h_ref[1]
    w = w_ref[...]
    b = b_ref[...]
    for j in range(6):
        a = jnp.dot(h0, w[j], preferred_element_type=_f32) + b[j]
        bb = jnp.dot(h1, w[j], preferred_element_type=_f32) + b[j]
        out_refs[j][0] = jnp.concatenate([a, bb], axis=1)


def _run_proj(h, w6, b6):
    nb = 2000
    grid = (T2, N // nb)
    return pl.pallas_call(
        _proj_body,
        grid=grid,
        in_specs=[
            pl.BlockSpec((2, nb, H), lambda t, i: (t, i, 0)),
            pl.BlockSpec((6, H, H), lambda t, i: (0, 0, 0)),
            pl.BlockSpec((6, 1, H), lambda t, i: (0, 0, 0)),
        ],
        out_specs=[pl.BlockSpec((1, nb, 2 * H), lambda t, i: (t, i, 0))] * 6,
        out_shape=[jax.ShapeDtypeStruct((T2, N, 2 * H), _f32)] * 6,
    )(h, w6, b6)


def _edge_body(fw_ref, fi_ref, fc_ref, w_ref, b_ref, ow_ref, oi_ref, oc_ref):
    w = w_ref[...]
    b = b_ref[...]
    for g, (fr, orf) in enumerate(((fw_ref, ow_ref), (fi_ref, oi_ref),
                                   (fc_ref, oc_ref))):
        orf[...] = jnp.dot(fr[...], w[g], preferred_element_type=_f32) + b[g]


def _run_edge(fw, fi, fc, w3, b3):
    eb = 2000
    grid = (E // eb,)
    fspec = pl.BlockSpec((eb, fw.shape[1]), lambda i: (i, 0))
    ospec = pl.BlockSpec((eb, H), lambda i: (i, 0))
    return pl.pallas_call(
        _edge_body,
        grid=grid,
        in_specs=[
            fspec, fspec, fspec,
            pl.BlockSpec((3, fw.shape[1], H), lambda i: (0, 0, 0)),
            pl.BlockSpec((3, 1, H), lambda i: (0, 0, 0)),
        ],
        out_specs=[ospec] * 3,
        out_shape=[jax.ShapeDtypeStruct((E, H), _f32)] * 3,
    )(fw, fi, fc, w3, b3)


RING = 4
IRING = 8


def _sc_body(eq, ek, e, src, dst, zz, out,
             acc, srcv, dstv, eqr, ekr, er, semq, semk, seme, semsc, semi):
    c = lax.axis_index("c")
    s = lax.axis_index("s")
    r_main = (N // NSUB) // 8 * 8
    r_last = N - (NSUB - 1) * r_main
    chunks_per_core = (E // CHUNK) // NCORE
    iters = (chunks_per_core + NSUB - 1) // NSUB
    iters = (iters + IRING - 1) // IRING * IRING

    def _rowcopy(src_ref, dst_ref):
        @pl.when(s < NSUB - 1)
        def _():
            pltpu.sync_copy(src_ref.at[pl.ds(s * r_main, r_main)],
                            dst_ref.at[pl.ds(s * r_main, r_main)])

        @pl.when(s == NSUB - 1)
        def _():
            base = (NSUB - 1) * r_main
            pltpu.sync_copy(src_ref.at[pl.ds(base, r_last)],
                            dst_ref.at[pl.ds(base, r_last)])

    def _ch(n):
        return s + n * NSUB

    def _valid(n):
        return _ch(n) < chunks_per_core

    def _base(n):
        return (c * chunks_per_core + _ch(n)) * CHUNK

    def _idx_issue(n, ji):
        @pl.when(_valid(n))
        def _():
            pltpu.async_copy(src.at[pl.ds(_base(n), CHUNK)], srcv.at[ji],
                             semi[ji])
            pltpu.async_copy(dst.at[pl.ds(_base(n), CHUNK)], dstv.at[ji],
                             semi[ji])

    def _gather_issue(n, b, be, ji, tp):
        @pl.when(_valid(n))
        def _():
            pltpu.make_async_copy(src.at[pl.ds(0, CHUNK)], srcv.at[ji],
                                  semi[ji]).wait()
            pltpu.make_async_copy(dst.at[pl.ds(0, CHUNK)], dstv.at[ji],
                                  semi[ji]).wait()
            pltpu.async_copy(e.at[pl.ds(_base(n), CHUNK)], er.at[be],
                             seme[be])
            pltpu.async_copy(eq.at[tp].at[dstv.at[ji]], eqr.at[b], semq[b])
            pltpu.async_copy(ek.at[tp].at[srcv.at[ji]], ekr.at[b], semk[b])

    def _scat_wait(b, ji):
        pltpu.make_async_copy(eqr.at[b], acc.at[dstv.at[ji]], semsc[b]).wait()

    def _compute_scatter(n, b, be, ji, tp):
        @pl.when(_valid(n))
        def _():
            pltpu.make_async_copy(eq.at[tp].at[dstv.at[ji]], eqr.at[b],
                                  semq[b]).wait()
            pltpu.make_async_copy(ek.at[tp].at[srcv.at[ji]], ekr.at[b],
                                  semk[b]).wait()
            pltpu.make_async_copy(e.at[pl.ds(0, CHUNK)], er.at[be],
                                  seme[be]).wait()

            @plsc.parallel_loop(0, CHUNK, unroll=2)
            def row_body(r):
                for cc in range(H // 16):
                    ev = er[be, r, pl.ds(cc * 16, 16)]
                    for u in range(2):
                        sl = pl.ds(u * H + cc * 16, 16)
                        v = eqr[b, r, sl] + ekr[b, r, sl] + ev
                        eqr[b, r, sl] = jnp.maximum(v, 0.2 * v)

            pltpu.async_copy(eqr.at[b], acc.at[dstv.at[ji]], semsc[b],
                             add=True)

    def tp_body(tp, _):
        _rowcopy(zz, acc)
        plsc.subcore_barrier()

        for j in range(RING):
            _idx_issue(j, j)
        for j in range(2):
            _gather_issue(j, j, j, j, tp)

        def octet_body(q, _, tp=tp):
            for k in range(IRING):
                n = q * IRING + k
                _idx_issue(n + 4, (k + 4) % IRING)
                _compute_scatter(n, k % RING, k % 2, k, tp)
                @pl.when((n >= 2) & _valid(n + 2))
                def _(n=n, k=k):
                    _scat_wait((k - 2) % RING, (k - 2) % IRING)
                _gather_issue(n + 2, (k + 2) % RING, k % 2, (k + 2) % IRING,
                              tp)
            return 0

        lax.fori_loop(0, iters // IRING, octet_body, 0)

        for k in range(IRING):
            n = iters - IRING + k

            @pl.when(_valid(n) & jnp.logical_not(_valid(n + 4)))
            def _(n=n, k=k):
                _scat_wait(k % RING, k % IRING)

        plsc.subcore_barrier()
        _rowcopy(acc, out.at[c].at[tp])
        plsc.subcore_barrier()
        return 0

    lax.fori_loop(0, T2, tp_body, 0)


def _make_sc_kernel():
    mesh = plsc.VectorSubcoreMesh(core_axis_name="c", subcore_axis_name="s",
                                  num_cores=NCORE, num_subcores=NSUB)
    return pl.kernel(
        _sc_body,
        out_type=jax.ShapeDtypeStruct((NCORE, T2, N, 2 * H), _f32),
        mesh=mesh,
        scratch_types=[
            pltpu.VMEM_SHARED((N, 2 * H), _f32),
            pltpu.VMEM((IRING, CHUNK), jnp.int32),
            pltpu.VMEM((IRING, CHUNK), jnp.int32),
            pltpu.VMEM((RING, CHUNK, 2 * H), _f32),
            pltpu.VMEM((RING, CHUNK, 2 * H), _f32),
            pltpu.VMEM((2, CHUNK, H), _f32),
            [pltpu.SemaphoreType.DMA] * RING,
            [pltpu.SemaphoreType.DMA] * RING,
            [pltpu.SemaphoreType.DMA] * 2,
            [pltpu.SemaphoreType.DMA] * RING,
            [pltpu.SemaphoreType.DMA] * IRING,
        ],
    )


def _readout_body(h_ref, fw_ref, fi_ref, fc_ref, xn0_ref,
                  wr_ref, br_ref, wa_ref, ba_ref, w1_ref, b1_ref,
                  w2_ref, b2_ref, out_ref):
    h = h_ref[...]
    sources = [[h[t] for t in range(T)]]
    for g, fref in enumerate((fw_ref, fi_ref, fc_ref)):
        f = fref[...]
        ft = f[0] + f[1]
        wr = wr_ref[g]
        br = br_ref[g]
        sources.append([
            jnp.dot(ft[t // 2][:, (t % 2) * H:(t % 2 + 1) * H], wr,
                    preferred_element_type=_f32) + br
            for t in range(T)])
    wa = wa_ref[...]
    ba = ba_ref[...]
    logit_cols = []
    for t in range(T):
        v = ba
        for p in range(4):
            v = v + jnp.dot(sources[p][t], wa[p * H:(p + 1) * H, :],
                            preferred_element_type=_f32)
        logit_cols.append(v)
    logits = jnp.concatenate(logit_cols, axis=1)
    m = jnp.max(logits, axis=1, keepdims=True)
    ex = jnp.exp(logits - m)
    w = ex / jnp.sum(ex, axis=1, keepdims=True)
    w1 = w1_ref[...]
    r = b1_ref[...]
    for p in range(4):
        pooled = sources[p][0] * w[:, 0:1]
        for t in range(1, T):
            pooled = pooled + sources[p][t] * w[:, t:t + 1]
        r = r + jnp.dot(pooled, w1[p * H:(p + 1) * H, :],
                        preferred_element_type=_f32)
    r = _leaky(r)
    r2 = _leaky(jnp.dot(r, w2_ref[...], preferred_element_type=_f32)
                + b2_ref[...])
    out_ref[...] = r2 / xn0_ref[...] - 1.0


def _run_readout(h, ftw, fti, ftc, xn0, wr3, br3, wa, ba, w1, b1, w2, b2):
    nb = 200
    grid = (N // nb,)
    fspec = pl.BlockSpec((NCORE, T2, nb, 2 * H), lambda i: (0, 0, i, 0))
    return pl.pallas_call(
        _readout_body,
        grid=grid,
        in_specs=[
            pl.BlockSpec((T, nb, H), lambda i: (0, i, 0)),
            fspec, fspec, fspec,
            pl.BlockSpec((nb, 1), lambda i: (i, 0)),
            pl.BlockSpec((3, H, H), lambda i: (0, 0, 0)),
            pl.BlockSpec((3, 1, H), lambda i: (0, 0, 0)),
            pl.BlockSpec((4 * H, 1), lambda i: (0, 0)),
            pl.BlockSpec((1, 1), lambda i: (0, 0)),
            pl.BlockSpec((4 * H, H), lambda i: (0, 0)),
            pl.BlockSpec((1, H), lambda i: (0, 0)),
            pl.BlockSpec((H, 1), lambda i: (0, 0)),
            pl.BlockSpec((1, 1), lambda i: (0, 0)),
        ],
        out_specs=pl.BlockSpec((nb, 1), lambda i: (i, 0)),
        out_shape=jax.ShapeDtypeStruct((N, 1), _f32),
    )(h, ftw, fti, ftc, xn0, wr3, br3, wa, ba, w1, b1, w2, b2)


def kernel(stock_features, wiki_edge_index, wiki_efeat, industry_edge_index,
           industry_efeat, correlation_edge_index, correlation_efeat,
           lstm_params, wiki_params, industry_params, corr_params,
           attn_params, readout_params):
    xf = stock_features.reshape(N, T * D_IN)
    wih = lstm_params["W_ih"].T
    whh = lstm_params["W_hh"].T
    b = (lstm_params["b_ih"] + lstm_params["b_hh"])[None, :]
    h, xn0 = _run_lstm(xf, wih, whh, b)

    gp = (wiki_params, industry_params, corr_params)
    wqk = jnp.stack([p[k].T for p in gp for k in ("Wq", "Wk")])
    bqk = jnp.stack([p[k][None, :] for p in gp for k in ("bq", "bk")])
    eqw, ekw, eqi, eki, eqc, ekc = _run_proj(h, wqk, bqk)

    we3 = jnp.stack([p["We"].T for p in gp])
    be3 = jnp.stack([p["be"][None, :] for p in gp])
    ew, ei, ec = _run_edge(wiki_efeat, industry_efeat, correlation_efeat,
                           we3, be3)

    zz = jnp.zeros((N, 2 * H), _f32)
    sc = _make_sc_kernel()
    ftw = sc(eqw, ekw, ew, wiki_edge_index[0], wiki_edge_index[1], zz)
    fti = sc(eqi, eki, ei, industry_edge_index[0], industry_edge_index[1], zz)
    ftc = sc(eqc, ekc, ec, correlation_edge_index[0],
             correlation_edge_index[1], zz)

    wr3 = jnp.stack([p["Wr"].T for p in gp])
    br3 = jnp.stack([p["br"][None, :] for p in gp])
    return _run_readout(
        h, ftw, fti, ftc, xn0, wr3, br3,
        attn_params["Wa"].T, attn_params["ba"][None, :],
        readout_params["W1"].T, readout_params["b1"][None, :],
        readout_params["W2"].T, readout_params["b2"][None, :])
